# Initial kernel scaffold; baseline (speedup 1.0000x reference)
import jax
import jax.numpy as jnp
from jax.experimental import pallas as pl


def kernel(log_probs, trg):
    raise NotImplementedError("write your pallas kernel here")


# speedup vs baseline: 6.9570x; 6.9570x over previous
import functools
import math

import jax
import jax.numpy as jnp
from jax.experimental import pallas as pl
from jax.experimental.pallas import tpu as pltpu

PAD = 1
SMOOTH = 0.1
ROWS = 2048
V = 32000
VB = 1280
NV = V // VB


def _xent_block(lp_ref, t_ref, out_ref, *, s, c):
    j = pl.program_id(0)
    t = t_ref[:, :]
    nonpad = (t != PAD).astype(jnp.float32)

    @pl.when(j == 0)
    def _init():
        out_ref[0, 0] = c * jnp.sum(nonpad)

    cols = j * VB + jax.lax.broadcasted_iota(jnp.int32, (1, VB), 1)
    w = (
        s
        + (1.0 - SMOOTH - s) * (cols == t).astype(jnp.float32)
        - s * (cols == PAD).astype(jnp.float32)
    )
    contrib = jnp.sum(lp_ref[:, :] * (w * nonpad))
    out_ref[0, 0] -= contrib


def kernel(log_probs, trg):
    s = SMOOTH / (V - 2)
    c = (1.0 - SMOOTH) * math.log(1.0 - SMOOTH) + SMOOTH * math.log(s)
    lp = log_probs.reshape(ROWS, V)
    t2 = trg.reshape(ROWS, 1)
    out = pl.pallas_call(
        functools.partial(_xent_block, s=s, c=c),
        grid=(NV,),
        in_specs=[
            pl.BlockSpec((ROWS, VB), lambda j: (0, j)),
            pl.BlockSpec((ROWS, 1), lambda j: (0, 0)),
        ],
        out_specs=pl.BlockSpec((1, 1), lambda j: (0, 0), memory_space=pltpu.MemorySpace.SMEM),
        out_shape=jax.ShapeDtypeStruct((1, 1), jnp.float32),
        compiler_params=pltpu.CompilerParams(
            dimension_semantics=("arbitrary",),
        ),
    )(lp, t2)
    return out[0, 0]
